# baseline (speedup 1.0000x reference)
import functools

import numpy as np
import jax
import jax.numpy as jnp
from jax import lax
from jax.experimental import pallas as pl
from jax.experimental.pallas import tpu as pltpu
from jax.experimental.pallas import tpu_sc as plsc

_NC = 2
_NS = 16
_NW = _NC * _NS
_L = 16

_U32 = np.uint32
_LOF = np.float32(np.nextafter(np.float32(-1.0), np.float32(0.0)))
_SPAN = np.float32(np.float32(1.0) - _LOF)
_SQRT2 = np.float32(np.sqrt(2.0))
_THIRD = np.float32(1.0 / 3.0)
_SQUEEZE = np.float32(0.0331)


def _rotl(x, r):
    return (x << _U32(r)) | (x >> _U32(32 - r))


def _tf(k1, k2, x0, x1):
    ks2 = k1 ^ k2 ^ _U32(0x1BD11BDA)
    x0 = x0 + k1
    x1 = x1 + k2
    R0 = (13, 15, 26, 6)
    R1 = (17, 29, 16, 24)

    def rounds(a, b, rs):
        for r in rs:
            a = a + b
            b = _rotl(b, r)
            b = b ^ a
        return a, b

    x0, x1 = rounds(x0, x1, R0)
    x0 = x0 + k2
    x1 = x1 + ks2 + _U32(1)
    x0, x1 = rounds(x0, x1, R1)
    x0 = x0 + ks2
    x1 = x1 + k1 + _U32(2)
    x0, x1 = rounds(x0, x1, R0)
    x0 = x0 + k1
    x1 = x1 + k2 + _U32(3)
    x0, x1 = rounds(x0, x1, R1)
    x0 = x0 + k2
    x1 = x1 + ks2 + _U32(4)
    x0, x1 = rounds(x0, x1, R0)
    x0 = x0 + ks2
    x1 = x1 + k1 + _U32(5)
    return x0, x1


def _tf_np(k1, k2, x0, x1):
    with np.errstate(over="ignore"):
        return _tf(np.asarray(k1, _U32), np.asarray(k2, _U32),
                   np.asarray(x0, _U32), np.asarray(x1, _U32))


@functools.lru_cache(maxsize=None)
def _derive_keys_np(seed, n):
    k1 = _U32(np.uint64(seed) >> np.uint64(32))
    k2 = _U32(np.uint64(seed) & np.uint64(0xFFFFFFFF))
    t1, t2 = _tf_np(k1, k2, _U32(0), _U32(0))
    e = np.arange(n, dtype=_U32)
    z = np.zeros(n, dtype=_U32)
    e1, e2 = _tf_np(np.full(n, t1, _U32), np.full(n, t2, _U32), z, e)
    m1, m2 = _tf_np(e1, e2, z, z)
    return m1, m2


def _bits_to_unit(bits):
    fb = (bits >> _U32(9)) | _U32(0x3F800000)
    return lax.bitcast_convert_type(fb, jnp.float32) - jnp.float32(1.0)


def _normal_from_key(s1, s2):
    h1, h2 = _tf(s1, s2, jnp.zeros_like(s1), jnp.zeros_like(s1))
    f = _bits_to_unit(h1 ^ h2)
    u = jnp.maximum(_LOF, f * _SPAN + _LOF)
    return _SQRT2 * lax.erf_inv(u)


def _uniform_from_key(s1, s2):
    h1, h2 = _tf(s1, s2, jnp.zeros_like(s1), jnp.zeros_like(s1))
    f = _bits_to_unit(h1 ^ h2)
    return jnp.maximum(jnp.float32(0.0), f)


def _gamma_core_dc(k1, k2, d, c):
    zero_u = jnp.zeros_like(k1)
    one_u = jnp.full_like(k1, 1)
    two_u = jnp.full_like(k1, 2)
    f1 = jnp.float32(1.0)

    def cond_el(X, V, U):
        return (U >= f1 - _SQUEEZE * X * X) & (
            jnp.log(U) >= jnp.float32(0.5) * X + d * (f1 - V + jnp.log(V)))

    def outer_cond(carry):
        _, _, X, V, U = carry
        return jnp.any(cond_el(X, V, U))

    def outer_body(carry):
        a1, a2, X, V, U = carry
        n1, n2 = _tf(a1, a2, zero_u, zero_u)
        xk1, xk2 = _tf(a1, a2, zero_u, one_u)
        uk1, uk2 = _tf(a1, a2, zero_u, two_u)

        def inner_cond(ic):
            _, _, _, v = ic
            return jnp.any(v <= jnp.float32(0.0))

        def inner_body(ic):
            b1, b2, x, v = ic
            nb1, nb2 = _tf(b1, b2, zero_u, zero_u)
            s1, s2 = _tf(b1, b2, zero_u, one_u)
            xn = _normal_from_key(s1, s2)
            vn = f1 + xn * c
            p = v <= jnp.float32(0.0)
            return (jnp.where(p, nb1, b1), jnp.where(p, nb2, b2),
                    jnp.where(p, xn, x), jnp.where(p, vn, v))

        _, _, x, v = lax.while_loop(
            inner_cond, inner_body,
            (xk1, xk2, jnp.zeros_like(d), jnp.full_like(d, -1.0)))

        Xn = x * x
        Vn = v * v * v
        Un = _uniform_from_key(uk1, uk2)
        p = cond_el(X, V, U)
        return (jnp.where(p, n1, a1), jnp.where(p, n2, a2),
                jnp.where(p, Xn, X), jnp.where(p, Vn, V), jnp.where(p, Un, U))

    init = (k1, k2, jnp.zeros_like(d), jnp.ones_like(d),
            jnp.full_like(d, 2.0))
    _, _, _, V, _ = lax.while_loop(outer_cond, outer_body, init)
    return d * V


@functools.lru_cache(maxsize=None)
def _make_seg_reduce(B, N, D, K):
    PTS = B * N // _NW
    GROUPS = PTS // _L
    ACC = 3 * K * D

    QPB = N // PTS

    mesh = plsc.VectorSubcoreMesh(
        core_axis_name="c", subcore_axis_name="s",
        num_cores=_NC, num_subcores=_NS)

    NCH = 4
    CPTS = PTS // NCH
    CGRP = GROUPS // NCH

    @functools.partial(
        pl.kernel,
        out_type=jax.ShapeDtypeStruct((_NW, ACC), jnp.float32),
        mesh=mesh,
        compiler_params=pltpu.CompilerParams(needs_layout_passes=False),
        scratch_types=(
            pltpu.VMEM((PTS, D), jnp.float32),
            pltpu.VMEM((PTS,), jnp.int32),
            pltpu.VMEM((ACC,), jnp.float32),
            tuple(pltpu.SemaphoreType.DMA for _ in range(NCH)),
            pltpu.SemaphoreType.DMA,
        ),
    )
    def seg_reduce(xs_hbm, zs_hbm, out_hbm, xs_v, z_v, acc, semx, semz):
        wid = lax.axis_index("c") * _NS + lax.axis_index("s")
        b = wid // QPB
        q = wid % QPB
        cps = [
            pltpu.async_copy(
                xs_hbm.at[b, pl.ds(q * PTS + ch * CPTS, CPTS), :],
                xs_v.at[pl.ds(ch * CPTS, CPTS), :], semx[ch])
            for ch in range(NCH)
        ]
        cpz = pltpu.async_copy(zs_hbm.at[pl.ds(wid * PTS, PTS)], z_v, semz)

        zero = jnp.zeros((_L,), jnp.float32)
        for i in range(ACC // _L):
            acc[pl.ds(i * _L, _L)] = zero

        cpz.wait()

        iota = lax.iota(jnp.int32, _L)
        ones = jnp.ones((_L,), jnp.float32)

        def group(g, carry):
            zi = z_v[pl.ds(g * _L, _L)] * D
            for j in range(_L):
                zsp = jnp.take_along_axis(
                    zi, jnp.full((_L,), j, jnp.int32), axis=0,
                    mode="promise_in_bounds")
                idx = zsp + iota
                xrow = xs_v[g * _L + j, :]
                plsc.addupdate_scatter(acc, [idx], xrow)
                plsc.addupdate_scatter(acc, [idx + (K * D)], xrow * xrow)
                plsc.addupdate_scatter(acc, [idx + (2 * K * D)], ones)
            return carry

        for ch in range(NCH):
            cps[ch].wait()
            lax.fori_loop(ch * CGRP, (ch + 1) * CGRP, group, 0, unroll=False)

        pltpu.sync_copy(acc, out_hbm.at[wid])

    return seg_reduce


@functools.lru_cache(maxsize=None)
def _make_stats(B, K, D):
    KD = K * D
    G = _NW // B

    def body(p_ref, c0_ref, r0_ref, conc_ref, a_ref, mu_ref):
        p = p_ref[...]
        s = jnp.sum(p[:, 0:KD].reshape(B, G, KD), axis=1)
        q = jnp.sum(p[:, KD:2 * KD].reshape(B, G, KD), axis=1)
        cc = jnp.sum(p[:, 2 * KD:3 * KD].reshape(B, G, KD), axis=1)
        nks = cc + 1.0
        m = s / nks
        sqdev = q - 2.0 * m * s + cc * m * m
        conc_ref[...] = c0_ref[...] + nks * 0.5
        rate = r0_ref[...] + sqdev * 0.5 + nks * m * m / (2.0 * (nks + 1.0))
        a_ref[...] = rate / (nks + 1.0)
        mu_ref[...] = s / (nks + 1.0)

    shp = jax.ShapeDtypeStruct((B, KD), jnp.float32)
    return pl.pallas_call(body, out_shape=(shp, shp, shp))


@functools.lru_cache(maxsize=None)
def _make_gamma_final(B, KD):
    def body(m1_ref, m2_ref, d_ref, c_ref, a_ref, mu_ref, eps_ref, out_ref):
        graw = _gamma_core_dc(m1_ref[...], m2_ref[...], d_ref[...], c_ref[...])
        out_ref[...] = (mu_ref[...]
                        + jnp.sqrt(a_ref[...] / graw) * eps_ref[...])

    return pl.pallas_call(
        body, out_shape=jax.ShapeDtypeStruct((B, KD), jnp.float32))


def kernel(xs, concentration0, rate0, zs):
    B, N, D = xs.shape
    K = concentration0.shape[0]
    KD = K * D

    zs_flat = zs.reshape(-1).astype(jnp.int32)
    partials = _make_seg_reduce(B, N, D, K)(xs, zs_flat)

    conc, a, mu = _make_stats(B, K, D)(
        partials, concentration0.reshape(1, KD), rate0.reshape(1, KD))

    d = conc - _THIRD
    c = _THIRD / lax.sqrt(d)

    m1, m2 = _derive_keys_np(42, B * KD)
    m1 = m1.reshape(B, KD)
    m2 = m2.reshape(B, KD)

    kn = jax.random.split(jax.random.key(42))[1]
    eps = jax.random.normal(kn, (B, K, D), dtype=xs.dtype).reshape(B, KD)

    mus = _make_gamma_final(B, KD)(m1, m2, d, c, a, mu, eps)
    return mus.reshape(B, K, D)


# speedup vs baseline: 1.0358x; 1.0358x over previous
import functools

import numpy as np
import jax
import jax.numpy as jnp
from jax import lax
from jax.experimental import pallas as pl
from jax.experimental.pallas import tpu as pltpu
from jax.experimental.pallas import tpu_sc as plsc

_NC = 2
_NS = 16
_NW = _NC * _NS
_L = 16

_U32 = np.uint32
_LOF = np.float32(np.nextafter(np.float32(-1.0), np.float32(0.0)))
_SPAN = np.float32(np.float32(1.0) - _LOF)
_SQRT2 = np.float32(np.sqrt(2.0))
_THIRD = np.float32(1.0 / 3.0)
_SQUEEZE = np.float32(0.0331)


def _rotl(x, r):
    return (x << _U32(r)) | (x >> _U32(32 - r))


def _tf(k1, k2, x0, x1):
    ks2 = k1 ^ k2 ^ _U32(0x1BD11BDA)
    x0 = x0 + k1
    x1 = x1 + k2
    R0 = (13, 15, 26, 6)
    R1 = (17, 29, 16, 24)

    def rounds(a, b, rs):
        for r in rs:
            a = a + b
            b = _rotl(b, r)
            b = b ^ a
        return a, b

    x0, x1 = rounds(x0, x1, R0)
    x0 = x0 + k2
    x1 = x1 + ks2 + _U32(1)
    x0, x1 = rounds(x0, x1, R1)
    x0 = x0 + ks2
    x1 = x1 + k1 + _U32(2)
    x0, x1 = rounds(x0, x1, R0)
    x0 = x0 + k1
    x1 = x1 + k2 + _U32(3)
    x0, x1 = rounds(x0, x1, R1)
    x0 = x0 + k2
    x1 = x1 + ks2 + _U32(4)
    x0, x1 = rounds(x0, x1, R0)
    x0 = x0 + ks2
    x1 = x1 + k1 + _U32(5)
    return x0, x1


def _tf_np(k1, k2, x0, x1):
    with np.errstate(over="ignore"):
        return _tf(np.asarray(k1, _U32), np.asarray(k2, _U32),
                   np.asarray(x0, _U32), np.asarray(x1, _U32))


_T_PRE = 6
_S_PRE = 3


@functools.lru_cache(maxsize=None)
def _derive_keys_np(seed, n):
    k1 = _U32(np.uint64(seed) >> np.uint64(32))
    k2 = _U32(np.uint64(seed) & np.uint64(0xFFFFFFFF))
    t1, t2 = _tf_np(k1, k2, _U32(0), _U32(0))
    e = np.arange(n, dtype=_U32)
    z = np.zeros(n, dtype=_U32)
    e1, e2 = _tf_np(np.full(n, t1, _U32), np.full(n, t2, _U32), z, e)
    m1, m2 = _tf_np(e1, e2, z, z)

    one = np.ones(n, dtype=_U32)
    two = np.full(n, 2, dtype=_U32)
    bits_x = np.empty((_T_PRE, _S_PRE, n), dtype=_U32)
    bits_u = np.empty((_T_PRE, n), dtype=_U32)
    ikey = np.empty((_T_PRE, 2, n), dtype=_U32)
    K1, K2 = m1, m2
    for t in range(_T_PRE):
        xk1, xk2 = _tf_np(K1, K2, z, one)
        uk1, uk2 = _tf_np(K1, K2, z, two)
        K1, K2 = _tf_np(K1, K2, z, z)
        b1, b2 = xk1, xk2
        for s in range(_S_PRE):
            s1, s2 = _tf_np(b1, b2, z, one)
            h1, h2 = _tf_np(s1, s2, z, z)
            bits_x[t, s] = h1 ^ h2
            b1, b2 = _tf_np(b1, b2, z, z)
        ikey[t, 0], ikey[t, 1] = b1, b2
        h1, h2 = _tf_np(uk1, uk2, z, z)
        bits_u[t] = h1 ^ h2
    fkey = np.stack([K1, K2])
    return m1, m2, bits_x, bits_u, ikey, fkey


def _bits_to_unit(bits):
    fb = (bits >> _U32(9)) | _U32(0x3F800000)
    return lax.bitcast_convert_type(fb, jnp.float32) - jnp.float32(1.0)


def _normal_from_key(s1, s2):
    h1, h2 = _tf(s1, s2, jnp.zeros_like(s1), jnp.zeros_like(s1))
    f = _bits_to_unit(h1 ^ h2)
    u = jnp.maximum(_LOF, f * _SPAN + _LOF)
    return _SQRT2 * lax.erf_inv(u)


def _uniform_from_key(s1, s2):
    h1, h2 = _tf(s1, s2, jnp.zeros_like(s1), jnp.zeros_like(s1))
    f = _bits_to_unit(h1 ^ h2)
    return jnp.maximum(jnp.float32(0.0), f)


def _normal_from_bits(bits):
    f = _bits_to_unit(bits)
    u = jnp.maximum(_LOF, f * _SPAN + _LOF)
    return _SQRT2 * lax.erf_inv(u)


def _cond_el(d, X, V, U):
    f1 = jnp.float32(1.0)
    return (U >= f1 - _SQUEEZE * X * X) & (
        jnp.log(U) >= jnp.float32(0.5) * X + d * (f1 - V + jnp.log(V)))


def _inner_fallback(b1, b2, x, v, c):
    f1 = jnp.float32(1.0)
    zero_u = jnp.zeros_like(b1)
    one_u = jnp.full_like(b1, 1)

    def inner_cond(ic):
        _, _, _, v_ = ic
        return jnp.any(v_ <= jnp.float32(0.0))

    def inner_body(ic):
        b1_, b2_, x_, v_ = ic
        nb1, nb2 = _tf(b1_, b2_, zero_u, zero_u)
        s1, s2 = _tf(b1_, b2_, zero_u, one_u)
        xn = _normal_from_key(s1, s2)
        vn = f1 + xn * c
        p = v_ <= jnp.float32(0.0)
        return (jnp.where(p, nb1, b1_), jnp.where(p, nb2, b2_),
                jnp.where(p, xn, x_), jnp.where(p, vn, v_))

    _, _, x, v = lax.while_loop(inner_cond, inner_body, (b1, b2, x, v))
    return x, v


def _gamma_loop(k1, k2, d, c, X, V, U):
    zero_u = jnp.zeros_like(k1)
    one_u = jnp.full_like(k1, 1)
    two_u = jnp.full_like(k1, 2)

    def outer_cond(carry):
        _, _, X_, V_, U_ = carry
        return jnp.any(_cond_el(d, X_, V_, U_))

    def outer_body(carry):
        a1, a2, X_, V_, U_ = carry
        n1, n2 = _tf(a1, a2, zero_u, zero_u)
        xk1, xk2 = _tf(a1, a2, zero_u, one_u)
        uk1, uk2 = _tf(a1, a2, zero_u, two_u)
        x, v = _inner_fallback(xk1, xk2, jnp.zeros_like(d),
                               jnp.full_like(d, -1.0), c)
        Xn = x * x
        Vn = v * v * v
        Un = _uniform_from_key(uk1, uk2)
        p = _cond_el(d, X_, V_, U_)
        return (jnp.where(p, n1, a1), jnp.where(p, n2, a2),
                jnp.where(p, Xn, X_), jnp.where(p, Vn, V_),
                jnp.where(p, Un, U_))

    _, _, _, V, _ = lax.while_loop(outer_cond, outer_body, (k1, k2, X, V, U))
    return V


def _gamma_fast(d, c, bx, bu, ik, fk1, fk2):
    f1 = jnp.float32(1.0)
    X = jnp.zeros_like(d)
    V = jnp.ones_like(d)
    U = jnp.full_like(d, 2.0)
    for t in range(_T_PRE):
        x = _normal_from_bits(bx[t, 0])
        v = f1 + x * c
        for s in range(1, _S_PRE):
            xs_ = _normal_from_bits(bx[t, s])
            vs_ = f1 + xs_ * c
            need = v <= jnp.float32(0.0)
            x = jnp.where(need, xs_, x)
            v = jnp.where(need, vs_, v)
        x, v = _inner_fallback(ik[t, 0], ik[t, 1], x, v, c)
        Xn = x * x
        Vn = v * v * v
        Un = jnp.maximum(jnp.float32(0.0), _bits_to_unit(bu[t]))
        p = _cond_el(d, X, V, U)
        X = jnp.where(p, Xn, X)
        V = jnp.where(p, Vn, V)
        U = jnp.where(p, Un, U)
    V = _gamma_loop(fk1, fk2, d, c, X, V, U)
    return d * V


@functools.lru_cache(maxsize=None)
def _make_seg_reduce(B, N, D, K):
    PTS = B * N // _NW
    GROUPS = PTS // _L
    ACC = 3 * K * D

    QPB = N // PTS

    mesh = plsc.VectorSubcoreMesh(
        core_axis_name="c", subcore_axis_name="s",
        num_cores=_NC, num_subcores=_NS)

    NCH = 4
    CPTS = PTS // NCH
    CGRP = GROUPS // NCH

    @functools.partial(
        pl.kernel,
        out_type=jax.ShapeDtypeStruct((_NW, ACC), jnp.float32),
        mesh=mesh,
        compiler_params=pltpu.CompilerParams(needs_layout_passes=False),
        scratch_types=(
            pltpu.VMEM((PTS, D), jnp.float32),
            pltpu.VMEM((PTS,), jnp.int32),
            pltpu.VMEM((ACC,), jnp.float32),
            pltpu.SemaphoreType.DMA,
            pltpu.SemaphoreType.DMA,
        ),
    )
    def seg_reduce(xs_hbm, zs_hbm, out_hbm, xs_v, z_v, acc, semx, semz):
        wid = lax.axis_index("c") * _NS + lax.axis_index("s")
        b = wid // QPB
        q = wid % QPB
        cpx = pltpu.async_copy(xs_hbm.at[b, pl.ds(q * PTS, PTS), :], xs_v, semx)
        cpz = pltpu.async_copy(zs_hbm.at[pl.ds(wid * PTS, PTS)], z_v, semz)

        zero = jnp.zeros((_L,), jnp.float32)
        for i in range(ACC // _L):
            acc[pl.ds(i * _L, _L)] = zero

        cpz.wait()

        iota = lax.iota(jnp.int32, _L)
        ones = jnp.ones((_L,), jnp.float32)

        def group(g, carry):
            zi = z_v[pl.ds(g * _L, _L)] * D
            for j in range(_L):
                zsp = jnp.take_along_axis(
                    zi, jnp.full((_L,), j, jnp.int32), axis=0,
                    mode="promise_in_bounds")
                idx = zsp + iota
                xrow = xs_v[g * _L + j, :]
                plsc.addupdate_scatter(acc, [idx], xrow)
                plsc.addupdate_scatter(acc, [idx + (K * D)], xrow * xrow)
                plsc.addupdate_scatter(acc, [idx + (2 * K * D)], ones)
            return carry

        cpx.wait()
        lax.fori_loop(0, GROUPS, group, 0, unroll=False)

        pltpu.sync_copy(acc, out_hbm.at[wid])

    return seg_reduce


@functools.lru_cache(maxsize=None)
def _make_stats(B, K, D):
    KD = K * D
    G = _NW // B

    def body(p_ref, c0_ref, r0_ref, conc_ref, a_ref, mu_ref):
        p = p_ref[...]
        s = jnp.sum(p[:, 0:KD].reshape(B, G, KD), axis=1)
        q = jnp.sum(p[:, KD:2 * KD].reshape(B, G, KD), axis=1)
        cc = jnp.sum(p[:, 2 * KD:3 * KD].reshape(B, G, KD), axis=1)
        nks = cc + 1.0
        m = s / nks
        sqdev = q - 2.0 * m * s + cc * m * m
        conc_ref[...] = c0_ref[...] + nks * 0.5
        rate = r0_ref[...] + sqdev * 0.5 + nks * m * m / (2.0 * (nks + 1.0))
        a_ref[...] = rate / (nks + 1.0)
        mu_ref[...] = s / (nks + 1.0)

    shp = jax.ShapeDtypeStruct((B, KD), jnp.float32)
    return pl.pallas_call(body, out_shape=(shp, shp, shp))


@functools.lru_cache(maxsize=None)
def _make_gamma_final(B, KD):
    def body(bx_ref, bu_ref, ik_ref, fk_ref, d_ref, c_ref, a_ref, mu_ref,
             eps_ref, out_ref):
        graw = _gamma_fast(d_ref[...], c_ref[...], bx_ref, bu_ref, ik_ref,
                           fk_ref[0], fk_ref[1])
        out_ref[...] = (mu_ref[...]
                        + jnp.sqrt(a_ref[...] / graw) * eps_ref[...])

    return pl.pallas_call(
        body, out_shape=jax.ShapeDtypeStruct((B, KD), jnp.float32))


def kernel(xs, concentration0, rate0, zs):
    B, N, D = xs.shape
    K = concentration0.shape[0]
    KD = K * D

    zs_flat = zs.reshape(-1).astype(jnp.int32)
    partials = _make_seg_reduce(B, N, D, K)(xs, zs_flat)

    conc, a, mu = _make_stats(B, K, D)(
        partials, concentration0.reshape(1, KD), rate0.reshape(1, KD))

    d = conc - _THIRD
    c = _THIRD / lax.sqrt(d)

    _, _, bits_x, bits_u, ikey, fkey = _derive_keys_np(42, B * KD)
    bx = bits_x.reshape(_T_PRE, _S_PRE, B, KD)
    bu = bits_u.reshape(_T_PRE, B, KD)
    ik = ikey.reshape(_T_PRE, 2, B, KD)
    fk = fkey.reshape(2, B, KD)

    kn = jax.random.split(jax.random.key(42))[1]
    eps = jax.random.normal(kn, (B, K, D), dtype=xs.dtype).reshape(B, KD)

    mus = _make_gamma_final(B, KD)(bx, bu, ik, fk, d, c, a, mu, eps)
    return mus.reshape(B, K, D)


# speedup vs baseline: 1.0551x; 1.0186x over previous
import functools

import numpy as np
import jax
import jax.numpy as jnp
from jax import lax
from jax.experimental import pallas as pl
from jax.experimental.pallas import tpu as pltpu
from jax.experimental.pallas import tpu_sc as plsc

_NC = 2
_NS = 16
_NW = _NC * _NS
_L = 16

_U32 = np.uint32
_LOF = np.float32(np.nextafter(np.float32(-1.0), np.float32(0.0)))
_SPAN = np.float32(np.float32(1.0) - _LOF)
_SQRT2 = np.float32(np.sqrt(2.0))
_THIRD = np.float32(1.0 / 3.0)
_SQUEEZE = np.float32(0.0331)


def _rotl(x, r):
    return (x << _U32(r)) | (x >> _U32(32 - r))


def _tf(k1, k2, x0, x1):
    ks2 = k1 ^ k2 ^ _U32(0x1BD11BDA)
    x0 = x0 + k1
    x1 = x1 + k2
    R0 = (13, 15, 26, 6)
    R1 = (17, 29, 16, 24)

    def rounds(a, b, rs):
        for r in rs:
            a = a + b
            b = _rotl(b, r)
            b = b ^ a
        return a, b

    x0, x1 = rounds(x0, x1, R0)
    x0 = x0 + k2
    x1 = x1 + ks2 + _U32(1)
    x0, x1 = rounds(x0, x1, R1)
    x0 = x0 + ks2
    x1 = x1 + k1 + _U32(2)
    x0, x1 = rounds(x0, x1, R0)
    x0 = x0 + k1
    x1 = x1 + k2 + _U32(3)
    x0, x1 = rounds(x0, x1, R1)
    x0 = x0 + k2
    x1 = x1 + ks2 + _U32(4)
    x0, x1 = rounds(x0, x1, R0)
    x0 = x0 + ks2
    x1 = x1 + k1 + _U32(5)
    return x0, x1


def _tf_np(k1, k2, x0, x1):
    with np.errstate(over="ignore"):
        return _tf(np.asarray(k1, _U32), np.asarray(k2, _U32),
                   np.asarray(x0, _U32), np.asarray(x1, _U32))


_T_PRE = 6
_S_PRE = 3


@functools.lru_cache(maxsize=None)
def _derive_keys_np(seed, n):
    k1 = _U32(np.uint64(seed) >> np.uint64(32))
    k2 = _U32(np.uint64(seed) & np.uint64(0xFFFFFFFF))
    t1, t2 = _tf_np(k1, k2, _U32(0), _U32(0))
    e = np.arange(n, dtype=_U32)
    z = np.zeros(n, dtype=_U32)
    e1, e2 = _tf_np(np.full(n, t1, _U32), np.full(n, t2, _U32), z, e)
    m1, m2 = _tf_np(e1, e2, z, z)

    one = np.ones(n, dtype=_U32)
    two = np.full(n, 2, dtype=_U32)
    bits_x = np.empty((_T_PRE, _S_PRE, n), dtype=_U32)
    bits_u = np.empty((_T_PRE, n), dtype=_U32)
    ikey = np.empty((_T_PRE, 2, n), dtype=_U32)
    K1, K2 = m1, m2
    for t in range(_T_PRE):
        xk1, xk2 = _tf_np(K1, K2, z, one)
        uk1, uk2 = _tf_np(K1, K2, z, two)
        K1, K2 = _tf_np(K1, K2, z, z)
        b1, b2 = xk1, xk2
        for s in range(_S_PRE):
            s1, s2 = _tf_np(b1, b2, z, one)
            h1, h2 = _tf_np(s1, s2, z, z)
            bits_x[t, s] = h1 ^ h2
            b1, b2 = _tf_np(b1, b2, z, z)
        ikey[t, 0], ikey[t, 1] = b1, b2
        h1, h2 = _tf_np(uk1, uk2, z, z)
        bits_u[t] = h1 ^ h2
    fkey = np.stack([K1, K2])
    return m1, m2, bits_x, bits_u, ikey, fkey


def _bits_to_unit(bits):
    fb = (bits >> _U32(9)) | _U32(0x3F800000)
    return lax.bitcast_convert_type(fb, jnp.float32) - jnp.float32(1.0)


def _normal_from_key(s1, s2):
    h1, h2 = _tf(s1, s2, jnp.zeros_like(s1), jnp.zeros_like(s1))
    f = _bits_to_unit(h1 ^ h2)
    u = jnp.maximum(_LOF, f * _SPAN + _LOF)
    return _SQRT2 * lax.erf_inv(u)


def _uniform_from_key(s1, s2):
    h1, h2 = _tf(s1, s2, jnp.zeros_like(s1), jnp.zeros_like(s1))
    f = _bits_to_unit(h1 ^ h2)
    return jnp.maximum(jnp.float32(0.0), f)


def _normal_from_bits(bits):
    f = _bits_to_unit(bits)
    u = jnp.maximum(_LOF, f * _SPAN + _LOF)
    return _SQRT2 * lax.erf_inv(u)


def _cond_el(d, X, V, U):
    f1 = jnp.float32(1.0)
    return (U >= f1 - _SQUEEZE * X * X) & (
        jnp.log(U) >= jnp.float32(0.5) * X + d * (f1 - V + jnp.log(V)))


def _inner_fallback(b1, b2, x, v, c):
    f1 = jnp.float32(1.0)
    zero_u = jnp.zeros_like(b1)
    one_u = jnp.full_like(b1, 1)

    def inner_cond(ic):
        _, _, _, v_ = ic
        return jnp.any(v_ <= jnp.float32(0.0))

    def inner_body(ic):
        b1_, b2_, x_, v_ = ic
        nb1, nb2 = _tf(b1_, b2_, zero_u, zero_u)
        s1, s2 = _tf(b1_, b2_, zero_u, one_u)
        xn = _normal_from_key(s1, s2)
        vn = f1 + xn * c
        p = v_ <= jnp.float32(0.0)
        return (jnp.where(p, nb1, b1_), jnp.where(p, nb2, b2_),
                jnp.where(p, xn, x_), jnp.where(p, vn, v_))

    _, _, x, v = lax.while_loop(inner_cond, inner_body, (b1, b2, x, v))
    return x, v


def _gamma_loop(k1, k2, d, c, X, V, U):
    zero_u = jnp.zeros_like(k1)
    one_u = jnp.full_like(k1, 1)
    two_u = jnp.full_like(k1, 2)

    def outer_cond(carry):
        _, _, X_, V_, U_ = carry
        return jnp.any(_cond_el(d, X_, V_, U_))

    def outer_body(carry):
        a1, a2, X_, V_, U_ = carry
        n1, n2 = _tf(a1, a2, zero_u, zero_u)
        xk1, xk2 = _tf(a1, a2, zero_u, one_u)
        uk1, uk2 = _tf(a1, a2, zero_u, two_u)
        x, v = _inner_fallback(xk1, xk2, jnp.zeros_like(d),
                               jnp.full_like(d, -1.0), c)
        Xn = x * x
        Vn = v * v * v
        Un = _uniform_from_key(uk1, uk2)
        p = _cond_el(d, X_, V_, U_)
        return (jnp.where(p, n1, a1), jnp.where(p, n2, a2),
                jnp.where(p, Xn, X_), jnp.where(p, Vn, V_),
                jnp.where(p, Un, U_))

    _, _, _, V, _ = lax.while_loop(outer_cond, outer_body, (k1, k2, X, V, U))
    return V


def _gamma_fast(d, c, bx, bu, ik, fk1, fk2):
    f1 = jnp.float32(1.0)
    X = jnp.zeros_like(d)
    V = jnp.ones_like(d)
    U = jnp.full_like(d, 2.0)
    for t in range(_T_PRE):
        x = _normal_from_bits(bx[t, 0])
        v = f1 + x * c
        for s in range(1, _S_PRE):
            xs_ = _normal_from_bits(bx[t, s])
            vs_ = f1 + xs_ * c
            need = v <= jnp.float32(0.0)
            x = jnp.where(need, xs_, x)
            v = jnp.where(need, vs_, v)
        x, v = _inner_fallback(ik[t, 0], ik[t, 1], x, v, c)
        Xn = x * x
        Vn = v * v * v
        Un = jnp.maximum(jnp.float32(0.0), _bits_to_unit(bu[t]))
        p = _cond_el(d, X, V, U)
        X = jnp.where(p, Xn, X)
        V = jnp.where(p, Vn, V)
        U = jnp.where(p, Un, U)
    V = _gamma_loop(fk1, fk2, d, c, X, V, U)
    return d * V


@functools.lru_cache(maxsize=None)
def _make_seg_reduce(B, N, D, K):
    PTS = B * N // _NW
    GROUPS = PTS // _L
    ACC = 3 * K * D

    QPB = N // PTS

    mesh = plsc.VectorSubcoreMesh(
        core_axis_name="c", subcore_axis_name="s",
        num_cores=_NC, num_subcores=_NS)

    NCH = 4
    CPTS = PTS // NCH
    CGRP = GROUPS // NCH

    @functools.partial(
        pl.kernel,
        out_type=jax.ShapeDtypeStruct((_NW, ACC), jnp.float32),
        mesh=mesh,
        compiler_params=pltpu.CompilerParams(needs_layout_passes=False),
        scratch_types=(
            pltpu.VMEM((PTS, D), jnp.float32),
            pltpu.VMEM((PTS,), jnp.int32),
            pltpu.VMEM((ACC,), jnp.float32),
            pltpu.SemaphoreType.DMA,
            pltpu.SemaphoreType.DMA,
        ),
    )
    def seg_reduce(xs_hbm, zs_hbm, out_hbm, xs_v, z_v, acc, semx, semz):
        wid = lax.axis_index("c") * _NS + lax.axis_index("s")
        b = wid // QPB
        q = wid % QPB
        cpx = pltpu.async_copy(xs_hbm.at[b, pl.ds(q * PTS, PTS), :], xs_v, semx)
        cpz = pltpu.async_copy(zs_hbm.at[b, pl.ds(q * PTS, PTS)], z_v, semz)

        zero = jnp.zeros((_L,), jnp.float32)
        for i in range(ACC // _L):
            acc[pl.ds(i * _L, _L)] = zero

        cpz.wait()

        iota = lax.iota(jnp.int32, _L)
        ones = jnp.ones((_L,), jnp.float32)

        def group(g, carry):
            zi = z_v[pl.ds(g * _L, _L)] * D
            for j in range(_L):
                zsp = jnp.take_along_axis(
                    zi, jnp.full((_L,), j, jnp.int32), axis=0,
                    mode="promise_in_bounds")
                idx = zsp + iota
                xrow = xs_v[g * _L + j, :]
                plsc.addupdate_scatter(acc, [idx], xrow)
                plsc.addupdate_scatter(acc, [idx + (K * D)], xrow * xrow)
                plsc.addupdate_scatter(acc, [idx + (2 * K * D)], ones)
            return carry

        cpx.wait()
        lax.fori_loop(0, GROUPS, group, 0, unroll=False)

        pltpu.sync_copy(acc, out_hbm.at[wid])

    return seg_reduce


@functools.lru_cache(maxsize=None)
def _make_stats(B, K, D):
    KD = K * D
    G = _NW // B

    def body(p_ref, c0_ref, r0_ref, conc_ref, a_ref, mu_ref):
        p = p_ref[...]
        s = jnp.sum(p[:, 0:KD].reshape(B, G, KD), axis=1)
        q = jnp.sum(p[:, KD:2 * KD].reshape(B, G, KD), axis=1)
        cc = jnp.sum(p[:, 2 * KD:3 * KD].reshape(B, G, KD), axis=1)
        nks = cc + 1.0
        m = s / nks
        sqdev = q - 2.0 * m * s + cc * m * m
        conc_ref[...] = c0_ref[...] + nks * 0.5
        rate = r0_ref[...] + sqdev * 0.5 + nks * m * m / (2.0 * (nks + 1.0))
        a_ref[...] = rate / (nks + 1.0)
        mu_ref[...] = s / (nks + 1.0)

    shp = jax.ShapeDtypeStruct((B, KD), jnp.float32)
    return pl.pallas_call(body, out_shape=(shp, shp, shp))


@functools.lru_cache(maxsize=None)
def _make_gamma_final(B, KD):
    def body(bx_ref, bu_ref, ik_ref, fk_ref, d_ref, c_ref, a_ref, mu_ref,
             eps_ref, out_ref):
        graw = _gamma_fast(d_ref[...], c_ref[...], bx_ref, bu_ref, ik_ref,
                           fk_ref[0], fk_ref[1])
        out_ref[...] = (mu_ref[...]
                        + jnp.sqrt(a_ref[...] / graw) * eps_ref[...])

    return pl.pallas_call(
        body, out_shape=jax.ShapeDtypeStruct((B, KD), jnp.float32))


def kernel(xs, concentration0, rate0, zs):
    B, N, D = xs.shape
    K = concentration0.shape[0]
    KD = K * D

    partials = _make_seg_reduce(B, N, D, K)(xs, zs.astype(jnp.int32))

    conc, a, mu = _make_stats(B, K, D)(
        partials, concentration0.reshape(1, KD), rate0.reshape(1, KD))

    d = conc - _THIRD
    c = _THIRD / lax.sqrt(d)

    _, _, bits_x, bits_u, ikey, fkey = _derive_keys_np(42, B * KD)
    bx = bits_x.reshape(_T_PRE, _S_PRE, B, KD)
    bu = bits_u.reshape(_T_PRE, B, KD)
    ik = ikey.reshape(_T_PRE, 2, B, KD)
    fk = fkey.reshape(2, B, KD)

    kn = jax.random.split(jax.random.key(42))[1]
    eps = jax.random.normal(kn, (B, K, D), dtype=xs.dtype).reshape(B, KD)

    mus = _make_gamma_final(B, KD)(bx, bu, ik, fk, d, c, a, mu, eps)
    return mus.reshape(B, K, D)


# speedup vs baseline: 1.0601x; 1.0048x over previous
import functools

import numpy as np
import jax
import jax.numpy as jnp
from jax import lax
from jax.experimental import pallas as pl
from jax.experimental.pallas import tpu as pltpu
from jax.experimental.pallas import tpu_sc as plsc

_NC = 2
_NS = 16
_NW = _NC * _NS
_L = 16

_U32 = np.uint32
_LOF = np.float32(np.nextafter(np.float32(-1.0), np.float32(0.0)))
_SPAN = np.float32(np.float32(1.0) - _LOF)
_SQRT2 = np.float32(np.sqrt(2.0))
_THIRD = np.float32(1.0 / 3.0)
_SQUEEZE = np.float32(0.0331)


def _rotl(x, r):
    return (x << _U32(r)) | (x >> _U32(32 - r))


def _tf(k1, k2, x0, x1):
    ks2 = k1 ^ k2 ^ _U32(0x1BD11BDA)
    x0 = x0 + k1
    x1 = x1 + k2
    R0 = (13, 15, 26, 6)
    R1 = (17, 29, 16, 24)

    def rounds(a, b, rs):
        for r in rs:
            a = a + b
            b = _rotl(b, r)
            b = b ^ a
        return a, b

    x0, x1 = rounds(x0, x1, R0)
    x0 = x0 + k2
    x1 = x1 + ks2 + _U32(1)
    x0, x1 = rounds(x0, x1, R1)
    x0 = x0 + ks2
    x1 = x1 + k1 + _U32(2)
    x0, x1 = rounds(x0, x1, R0)
    x0 = x0 + k1
    x1 = x1 + k2 + _U32(3)
    x0, x1 = rounds(x0, x1, R1)
    x0 = x0 + k2
    x1 = x1 + ks2 + _U32(4)
    x0, x1 = rounds(x0, x1, R0)
    x0 = x0 + ks2
    x1 = x1 + k1 + _U32(5)
    return x0, x1


def _tf_np(k1, k2, x0, x1):
    with np.errstate(over="ignore"):
        return _tf(np.asarray(k1, _U32), np.asarray(k2, _U32),
                   np.asarray(x0, _U32), np.asarray(x1, _U32))


_T_PRE = 6
_S_PRE = 3


@functools.lru_cache(maxsize=None)
def _derive_keys_np(seed, n):
    k1 = _U32(np.uint64(seed) >> np.uint64(32))
    k2 = _U32(np.uint64(seed) & np.uint64(0xFFFFFFFF))
    t1, t2 = _tf_np(k1, k2, _U32(0), _U32(0))
    e = np.arange(n, dtype=_U32)
    z = np.zeros(n, dtype=_U32)
    e1, e2 = _tf_np(np.full(n, t1, _U32), np.full(n, t2, _U32), z, e)
    m1, m2 = _tf_np(e1, e2, z, z)

    one = np.ones(n, dtype=_U32)
    two = np.full(n, 2, dtype=_U32)
    bits_x = np.empty((_T_PRE, _S_PRE, n), dtype=_U32)
    bits_u = np.empty((_T_PRE, n), dtype=_U32)
    ikey = np.empty((_T_PRE, 2, n), dtype=_U32)
    K1, K2 = m1, m2
    for t in range(_T_PRE):
        xk1, xk2 = _tf_np(K1, K2, z, one)
        uk1, uk2 = _tf_np(K1, K2, z, two)
        K1, K2 = _tf_np(K1, K2, z, z)
        b1, b2 = xk1, xk2
        for s in range(_S_PRE):
            s1, s2 = _tf_np(b1, b2, z, one)
            h1, h2 = _tf_np(s1, s2, z, z)
            bits_x[t, s] = h1 ^ h2
            b1, b2 = _tf_np(b1, b2, z, z)
        ikey[t, 0], ikey[t, 1] = b1, b2
        h1, h2 = _tf_np(uk1, uk2, z, z)
        bits_u[t] = h1 ^ h2
    fkey = np.stack([K1, K2])
    return m1, m2, bits_x, bits_u, ikey, fkey


def _bits_to_unit(bits):
    fb = (bits >> _U32(9)) | _U32(0x3F800000)
    return lax.bitcast_convert_type(fb, jnp.float32) - jnp.float32(1.0)


def _normal_from_key(s1, s2):
    h1, h2 = _tf(s1, s2, jnp.zeros_like(s1), jnp.zeros_like(s1))
    f = _bits_to_unit(h1 ^ h2)
    u = jnp.maximum(_LOF, f * _SPAN + _LOF)
    return _SQRT2 * lax.erf_inv(u)


def _uniform_from_key(s1, s2):
    h1, h2 = _tf(s1, s2, jnp.zeros_like(s1), jnp.zeros_like(s1))
    f = _bits_to_unit(h1 ^ h2)
    return jnp.maximum(jnp.float32(0.0), f)


def _normal_from_bits(bits):
    f = _bits_to_unit(bits)
    u = jnp.maximum(_LOF, f * _SPAN + _LOF)
    return _SQRT2 * lax.erf_inv(u)


def _cond_el(d, X, V, U):
    f1 = jnp.float32(1.0)
    return (U >= f1 - _SQUEEZE * X * X) & (
        jnp.log(U) >= jnp.float32(0.5) * X + d * (f1 - V + jnp.log(V)))


def _inner_fallback(b1, b2, x, v, c):
    f1 = jnp.float32(1.0)
    zero_u = jnp.zeros_like(b1)
    one_u = jnp.full_like(b1, 1)

    def inner_cond(ic):
        _, _, _, v_ = ic
        return jnp.any(v_ <= jnp.float32(0.0))

    def inner_body(ic):
        b1_, b2_, x_, v_ = ic
        nb1, nb2 = _tf(b1_, b2_, zero_u, zero_u)
        s1, s2 = _tf(b1_, b2_, zero_u, one_u)
        xn = _normal_from_key(s1, s2)
        vn = f1 + xn * c
        p = v_ <= jnp.float32(0.0)
        return (jnp.where(p, nb1, b1_), jnp.where(p, nb2, b2_),
                jnp.where(p, xn, x_), jnp.where(p, vn, v_))

    _, _, x, v = lax.while_loop(inner_cond, inner_body, (b1, b2, x, v))
    return x, v


def _gamma_loop(k1, k2, d, c, X, V, U):
    zero_u = jnp.zeros_like(k1)
    one_u = jnp.full_like(k1, 1)
    two_u = jnp.full_like(k1, 2)

    def outer_cond(carry):
        _, _, X_, V_, U_ = carry
        return jnp.any(_cond_el(d, X_, V_, U_))

    def outer_body(carry):
        a1, a2, X_, V_, U_ = carry
        n1, n2 = _tf(a1, a2, zero_u, zero_u)
        xk1, xk2 = _tf(a1, a2, zero_u, one_u)
        uk1, uk2 = _tf(a1, a2, zero_u, two_u)
        x, v = _inner_fallback(xk1, xk2, jnp.zeros_like(d),
                               jnp.full_like(d, -1.0), c)
        Xn = x * x
        Vn = v * v * v
        Un = _uniform_from_key(uk1, uk2)
        p = _cond_el(d, X_, V_, U_)
        return (jnp.where(p, n1, a1), jnp.where(p, n2, a2),
                jnp.where(p, Xn, X_), jnp.where(p, Vn, V_),
                jnp.where(p, Un, U_))

    _, _, _, V, _ = lax.while_loop(outer_cond, outer_body, (k1, k2, X, V, U))
    return V


def _gamma_fast(d, c, bx, bu, ik, fk1, fk2):
    f1 = jnp.float32(1.0)
    X = jnp.zeros_like(d)
    V = jnp.ones_like(d)
    U = jnp.full_like(d, 2.0)
    for t in range(_T_PRE):
        x = _normal_from_bits(bx[t, 0])
        v = f1 + x * c
        for s in range(1, _S_PRE):
            xs_ = _normal_from_bits(bx[t, s])
            vs_ = f1 + xs_ * c
            need = v <= jnp.float32(0.0)
            x = jnp.where(need, xs_, x)
            v = jnp.where(need, vs_, v)
        x, v = _inner_fallback(ik[t, 0], ik[t, 1], x, v, c)
        Xn = x * x
        Vn = v * v * v
        Un = jnp.maximum(jnp.float32(0.0), _bits_to_unit(bu[t]))
        p = _cond_el(d, X, V, U)
        X = jnp.where(p, Xn, X)
        V = jnp.where(p, Vn, V)
        U = jnp.where(p, Un, U)
    V = _gamma_loop(fk1, fk2, d, c, X, V, U)
    return d * V


@functools.lru_cache(maxsize=None)
def _make_seg_reduce(B, N, D, K):
    PTS = B * N // _NW
    GROUPS = PTS // _L
    ACC = 3 * K * D

    QPB = N // PTS

    mesh = plsc.VectorSubcoreMesh(
        core_axis_name="c", subcore_axis_name="s",
        num_cores=_NC, num_subcores=_NS)

    NCH = 4
    CPTS = PTS // NCH
    CGRP = GROUPS // NCH

    @functools.partial(
        pl.kernel,
        out_type=jax.ShapeDtypeStruct((_NW, ACC), jnp.float32),
        mesh=mesh,
        compiler_params=pltpu.CompilerParams(needs_layout_passes=False),
        scratch_types=(
            pltpu.VMEM((PTS, D), jnp.float32),
            pltpu.VMEM((PTS,), jnp.int32),
            pltpu.VMEM((ACC,), jnp.float32),
            pltpu.SemaphoreType.DMA,
            pltpu.SemaphoreType.DMA,
        ),
    )
    def seg_reduce(xs_hbm, zs_hbm, out_hbm, xs_v, z_v, acc, semx, semz):
        wid = lax.axis_index("c") * _NS + lax.axis_index("s")
        b = wid // QPB
        q = wid % QPB
        cpx = pltpu.async_copy(xs_hbm.at[b, pl.ds(q * PTS, PTS), :], xs_v, semx)
        cpz = pltpu.async_copy(zs_hbm.at[b, pl.ds(q * PTS, PTS)], z_v, semz)

        zero = jnp.zeros((_L,), jnp.float32)
        for i in range(ACC // _L):
            acc[pl.ds(i * _L, _L)] = zero

        cpz.wait()

        iota = lax.iota(jnp.int32, _L)
        ones = jnp.ones((_L,), jnp.float32)

        def group(g, carry):
            zi = z_v[pl.ds(g * _L, _L)] * D
            for j in range(_L):
                zsp = jnp.take_along_axis(
                    zi, jnp.full((_L,), j, jnp.int32), axis=0,
                    mode="promise_in_bounds")
                idx = zsp + iota
                xrow = xs_v[g * _L + j, :]
                plsc.addupdate_scatter(acc, [idx], xrow)
                plsc.addupdate_scatter(acc, [idx + (K * D)], xrow * xrow)
                plsc.addupdate_scatter(acc, [idx + (2 * K * D)], ones)
            return carry

        cpx.wait()
        lax.fori_loop(0, GROUPS, group, 0, unroll=2)

        pltpu.sync_copy(acc, out_hbm.at[wid])

    return seg_reduce


@functools.lru_cache(maxsize=None)
def _make_stats(B, K, D):
    KD = K * D
    G = _NW // B

    def body(p_ref, c0_ref, r0_ref, conc_ref, a_ref, mu_ref):
        p = p_ref[...]
        s = jnp.sum(p[:, 0:KD].reshape(B, G, KD), axis=1)
        q = jnp.sum(p[:, KD:2 * KD].reshape(B, G, KD), axis=1)
        cc = jnp.sum(p[:, 2 * KD:3 * KD].reshape(B, G, KD), axis=1)
        nks = cc + 1.0
        m = s / nks
        sqdev = q - 2.0 * m * s + cc * m * m
        conc_ref[...] = c0_ref[...] + nks * 0.5
        rate = r0_ref[...] + sqdev * 0.5 + nks * m * m / (2.0 * (nks + 1.0))
        a_ref[...] = rate / (nks + 1.0)
        mu_ref[...] = s / (nks + 1.0)

    shp = jax.ShapeDtypeStruct((B, KD), jnp.float32)
    return pl.pallas_call(body, out_shape=(shp, shp, shp))


@functools.lru_cache(maxsize=None)
def _make_gamma_final(B, KD):
    def body(bx_ref, bu_ref, ik_ref, fk_ref, d_ref, c_ref, a_ref, mu_ref,
             eps_ref, out_ref):
        graw = _gamma_fast(d_ref[...], c_ref[...], bx_ref, bu_ref, ik_ref,
                           fk_ref[0], fk_ref[1])
        out_ref[...] = (mu_ref[...]
                        + jnp.sqrt(a_ref[...] / graw) * eps_ref[...])

    return pl.pallas_call(
        body, out_shape=jax.ShapeDtypeStruct((B, KD), jnp.float32))


def kernel(xs, concentration0, rate0, zs):
    B, N, D = xs.shape
    K = concentration0.shape[0]
    KD = K * D

    partials = _make_seg_reduce(B, N, D, K)(xs, zs.astype(jnp.int32))

    conc, a, mu = _make_stats(B, K, D)(
        partials, concentration0.reshape(1, KD), rate0.reshape(1, KD))

    d = conc - _THIRD
    c = _THIRD / lax.sqrt(d)

    _, _, bits_x, bits_u, ikey, fkey = _derive_keys_np(42, B * KD)
    bx = bits_x.reshape(_T_PRE, _S_PRE, B, KD)
    bu = bits_u.reshape(_T_PRE, B, KD)
    ik = ikey.reshape(_T_PRE, 2, B, KD)
    fk = fkey.reshape(2, B, KD)

    kn = jax.random.split(jax.random.key(42))[1]
    eps = jax.random.normal(kn, (B, K, D), dtype=xs.dtype).reshape(B, KD)

    mus = _make_gamma_final(B, KD)(bx, bu, ik, fk, d, c, a, mu, eps)
    return mus.reshape(B, K, D)


# speedup vs baseline: 1.0610x; 1.0008x over previous
import functools

import numpy as np
import jax
import jax.numpy as jnp
from jax import lax
from jax.experimental import pallas as pl
from jax.experimental.pallas import tpu as pltpu
from jax.experimental.pallas import tpu_sc as plsc

_NC = 2
_NS = 16
_NW = _NC * _NS
_L = 16

_U32 = np.uint32
_LOF = np.float32(np.nextafter(np.float32(-1.0), np.float32(0.0)))
_SPAN = np.float32(np.float32(1.0) - _LOF)
_SQRT2 = np.float32(np.sqrt(2.0))
_THIRD = np.float32(1.0 / 3.0)
_SQUEEZE = np.float32(0.0331)


def _rotl(x, r):
    return (x << _U32(r)) | (x >> _U32(32 - r))


def _tf(k1, k2, x0, x1):
    ks2 = k1 ^ k2 ^ _U32(0x1BD11BDA)
    x0 = x0 + k1
    x1 = x1 + k2
    R0 = (13, 15, 26, 6)
    R1 = (17, 29, 16, 24)

    def rounds(a, b, rs):
        for r in rs:
            a = a + b
            b = _rotl(b, r)
            b = b ^ a
        return a, b

    x0, x1 = rounds(x0, x1, R0)
    x0 = x0 + k2
    x1 = x1 + ks2 + _U32(1)
    x0, x1 = rounds(x0, x1, R1)
    x0 = x0 + ks2
    x1 = x1 + k1 + _U32(2)
    x0, x1 = rounds(x0, x1, R0)
    x0 = x0 + k1
    x1 = x1 + k2 + _U32(3)
    x0, x1 = rounds(x0, x1, R1)
    x0 = x0 + k2
    x1 = x1 + ks2 + _U32(4)
    x0, x1 = rounds(x0, x1, R0)
    x0 = x0 + ks2
    x1 = x1 + k1 + _U32(5)
    return x0, x1


def _tf_np(k1, k2, x0, x1):
    with np.errstate(over="ignore"):
        return _tf(np.asarray(k1, _U32), np.asarray(k2, _U32),
                   np.asarray(x0, _U32), np.asarray(x1, _U32))


_T_PRE = 6
_S_PRE = 3


@functools.lru_cache(maxsize=None)
def _derive_keys_np(seed, n):
    k1 = _U32(np.uint64(seed) >> np.uint64(32))
    k2 = _U32(np.uint64(seed) & np.uint64(0xFFFFFFFF))
    t1, t2 = _tf_np(k1, k2, _U32(0), _U32(0))
    e = np.arange(n, dtype=_U32)
    z = np.zeros(n, dtype=_U32)
    e1, e2 = _tf_np(np.full(n, t1, _U32), np.full(n, t2, _U32), z, e)
    m1, m2 = _tf_np(e1, e2, z, z)

    one = np.ones(n, dtype=_U32)
    two = np.full(n, 2, dtype=_U32)
    bits_x = np.empty((_T_PRE, _S_PRE, n), dtype=_U32)
    bits_u = np.empty((_T_PRE, n), dtype=_U32)
    ikey = np.empty((_T_PRE, 2, n), dtype=_U32)
    K1, K2 = m1, m2
    for t in range(_T_PRE):
        xk1, xk2 = _tf_np(K1, K2, z, one)
        uk1, uk2 = _tf_np(K1, K2, z, two)
        K1, K2 = _tf_np(K1, K2, z, z)
        b1, b2 = xk1, xk2
        for s in range(_S_PRE):
            s1, s2 = _tf_np(b1, b2, z, one)
            h1, h2 = _tf_np(s1, s2, z, z)
            bits_x[t, s] = h1 ^ h2
            b1, b2 = _tf_np(b1, b2, z, z)
        ikey[t, 0], ikey[t, 1] = b1, b2
        h1, h2 = _tf_np(uk1, uk2, z, z)
        bits_u[t] = h1 ^ h2
    fkey = np.stack([K1, K2])
    return m1, m2, bits_x, bits_u, ikey, fkey


def _bits_to_unit(bits):
    fb = (bits >> _U32(9)) | _U32(0x3F800000)
    return lax.bitcast_convert_type(fb, jnp.float32) - jnp.float32(1.0)


def _normal_from_key(s1, s2):
    h1, h2 = _tf(s1, s2, jnp.zeros_like(s1), jnp.zeros_like(s1))
    f = _bits_to_unit(h1 ^ h2)
    u = jnp.maximum(_LOF, f * _SPAN + _LOF)
    return _SQRT2 * lax.erf_inv(u)


def _uniform_from_key(s1, s2):
    h1, h2 = _tf(s1, s2, jnp.zeros_like(s1), jnp.zeros_like(s1))
    f = _bits_to_unit(h1 ^ h2)
    return jnp.maximum(jnp.float32(0.0), f)


def _normal_from_bits(bits):
    f = _bits_to_unit(bits)
    u = jnp.maximum(_LOF, f * _SPAN + _LOF)
    return _SQRT2 * lax.erf_inv(u)


def _cond_el(d, X, V, U):
    f1 = jnp.float32(1.0)
    return (U >= f1 - _SQUEEZE * X * X) & (
        jnp.log(U) >= jnp.float32(0.5) * X + d * (f1 - V + jnp.log(V)))


def _inner_fallback(b1, b2, x, v, c):
    f1 = jnp.float32(1.0)
    zero_u = jnp.zeros_like(b1)
    one_u = jnp.full_like(b1, 1)

    def inner_cond(ic):
        _, _, _, v_ = ic
        return jnp.any(v_ <= jnp.float32(0.0))

    def inner_body(ic):
        b1_, b2_, x_, v_ = ic
        nb1, nb2 = _tf(b1_, b2_, zero_u, zero_u)
        s1, s2 = _tf(b1_, b2_, zero_u, one_u)
        xn = _normal_from_key(s1, s2)
        vn = f1 + xn * c
        p = v_ <= jnp.float32(0.0)
        return (jnp.where(p, nb1, b1_), jnp.where(p, nb2, b2_),
                jnp.where(p, xn, x_), jnp.where(p, vn, v_))

    _, _, x, v = lax.while_loop(inner_cond, inner_body, (b1, b2, x, v))
    return x, v


def _gamma_loop(k1, k2, d, c, X, V, U):
    zero_u = jnp.zeros_like(k1)
    one_u = jnp.full_like(k1, 1)
    two_u = jnp.full_like(k1, 2)

    def outer_cond(carry):
        _, _, X_, V_, U_ = carry
        return jnp.any(_cond_el(d, X_, V_, U_))

    def outer_body(carry):
        a1, a2, X_, V_, U_ = carry
        n1, n2 = _tf(a1, a2, zero_u, zero_u)
        xk1, xk2 = _tf(a1, a2, zero_u, one_u)
        uk1, uk2 = _tf(a1, a2, zero_u, two_u)
        x, v = _inner_fallback(xk1, xk2, jnp.zeros_like(d),
                               jnp.full_like(d, -1.0), c)
        Xn = x * x
        Vn = v * v * v
        Un = _uniform_from_key(uk1, uk2)
        p = _cond_el(d, X_, V_, U_)
        return (jnp.where(p, n1, a1), jnp.where(p, n2, a2),
                jnp.where(p, Xn, X_), jnp.where(p, Vn, V_),
                jnp.where(p, Un, U_))

    _, _, _, V, _ = lax.while_loop(outer_cond, outer_body, (k1, k2, X, V, U))
    return V


def _gamma_fast(d, c, bx, bu, ik, fk1, fk2):
    f1 = jnp.float32(1.0)
    X = jnp.zeros_like(d)
    V = jnp.ones_like(d)
    U = jnp.full_like(d, 2.0)
    for t in range(_T_PRE):
        x = _normal_from_bits(bx[t, 0])
        v = f1 + x * c
        for s in range(1, _S_PRE):
            xs_ = _normal_from_bits(bx[t, s])
            vs_ = f1 + xs_ * c
            need = v <= jnp.float32(0.0)
            x = jnp.where(need, xs_, x)
            v = jnp.where(need, vs_, v)
        x, v = _inner_fallback(ik[t, 0], ik[t, 1], x, v, c)
        Xn = x * x
        Vn = v * v * v
        Un = jnp.maximum(jnp.float32(0.0), _bits_to_unit(bu[t]))
        p = _cond_el(d, X, V, U)
        X = jnp.where(p, Xn, X)
        V = jnp.where(p, Vn, V)
        U = jnp.where(p, Un, U)
    V = _gamma_loop(fk1, fk2, d, c, X, V, U)
    return d * V


@functools.lru_cache(maxsize=None)
def _make_seg_reduce(B, N, D, K):
    PTS = B * N // _NW
    GROUPS = PTS // _L
    ACC = 3 * K * D

    QPB = N // PTS

    mesh = plsc.VectorSubcoreMesh(
        core_axis_name="c", subcore_axis_name="s",
        num_cores=_NC, num_subcores=_NS)

    @functools.partial(
        pl.kernel,
        out_type=jax.ShapeDtypeStruct((_NW, ACC), jnp.float32),
        mesh=mesh,
        compiler_params=pltpu.CompilerParams(needs_layout_passes=False),
        scratch_types=(
            pltpu.VMEM((PTS, D), jnp.float32),
            pltpu.VMEM((PTS,), jnp.int32),
            pltpu.VMEM((ACC,), jnp.float32),
            pltpu.SemaphoreType.DMA,
            pltpu.SemaphoreType.DMA,
        ),
    )
    def seg_reduce(xs_hbm, zs_hbm, out_hbm, xs_v, z_v, acc, semx, semz):
        wid = lax.axis_index("c") * _NS + lax.axis_index("s")
        b = wid // QPB
        q = wid % QPB
        cpx = pltpu.async_copy(xs_hbm.at[b, pl.ds(q * PTS, PTS), :], xs_v, semx)
        cpz = pltpu.async_copy(zs_hbm.at[b, pl.ds(q * PTS, PTS)], z_v, semz)

        zero = jnp.zeros((_L,), jnp.float32)
        for i in range(ACC // _L):
            acc[pl.ds(i * _L, _L)] = zero

        cpz.wait()

        iota = lax.iota(jnp.int32, _L)
        ones = jnp.ones((_L,), jnp.float32)

        def group(g, carry):
            zi = z_v[pl.ds(g * _L, _L)] * D
            for j in range(_L):
                zsp = jnp.take_along_axis(
                    zi, jnp.full((_L,), j, jnp.int32), axis=0,
                    mode="promise_in_bounds")
                idx = zsp + iota
                xrow = xs_v[g * _L + j, :]
                plsc.addupdate_scatter(acc, [idx], xrow)
                plsc.addupdate_scatter(acc, [idx + (K * D)], xrow * xrow)
                plsc.addupdate_scatter(acc, [idx + (2 * K * D)], ones)
            return carry

        cpx.wait()
        lax.fori_loop(0, GROUPS, group, 0, unroll=2)

        pltpu.sync_copy(acc, out_hbm.at[wid])

    return seg_reduce


@functools.lru_cache(maxsize=None)
def _make_stats(B, K, D):
    KD = K * D
    G = _NW // B

    def body(p_ref, c0_ref, r0_ref, conc_ref, a_ref, mu_ref):
        p = p_ref[...]
        s = jnp.sum(p[:, 0:KD].reshape(B, G, KD), axis=1)
        q = jnp.sum(p[:, KD:2 * KD].reshape(B, G, KD), axis=1)
        cc = jnp.sum(p[:, 2 * KD:3 * KD].reshape(B, G, KD), axis=1)
        nks = cc + 1.0
        m = s / nks
        sqdev = q - 2.0 * m * s + cc * m * m
        conc_ref[...] = c0_ref[...] + nks * 0.5
        rate = r0_ref[...] + sqdev * 0.5 + nks * m * m / (2.0 * (nks + 1.0))
        a_ref[...] = rate / (nks + 1.0)
        mu_ref[...] = s / (nks + 1.0)

    shp = jax.ShapeDtypeStruct((B, KD), jnp.float32)
    return pl.pallas_call(body, out_shape=(shp, shp, shp))


@functools.lru_cache(maxsize=None)
def _make_gamma_final(B, KD):
    def body(bx_ref, bu_ref, ik_ref, fk_ref, d_ref, c_ref, a_ref, mu_ref,
             eps_ref, out_ref):
        graw = _gamma_fast(d_ref[...], c_ref[...], bx_ref, bu_ref, ik_ref,
                           fk_ref[0], fk_ref[1])
        out_ref[...] = (mu_ref[...]
                        + jnp.sqrt(a_ref[...] / graw) * eps_ref[...])

    return pl.pallas_call(
        body, out_shape=jax.ShapeDtypeStruct((B, KD), jnp.float32))


def kernel(xs, concentration0, rate0, zs):
    B, N, D = xs.shape
    K = concentration0.shape[0]
    KD = K * D

    partials = _make_seg_reduce(B, N, D, K)(xs, zs.astype(jnp.int32))

    conc, a, mu = _make_stats(B, K, D)(
        partials, concentration0.reshape(1, KD), rate0.reshape(1, KD))

    d = conc - _THIRD
    c = _THIRD / lax.sqrt(d)

    _, _, bits_x, bits_u, ikey, fkey = _derive_keys_np(42, B * KD)
    bx = bits_x.reshape(_T_PRE, _S_PRE, B, KD)
    bu = bits_u.reshape(_T_PRE, B, KD)
    ik = ikey.reshape(_T_PRE, 2, B, KD)
    fk = fkey.reshape(2, B, KD)

    kn = jax.random.split(jax.random.key(42))[1]
    eps = jax.random.normal(kn, (B, K, D), dtype=xs.dtype).reshape(B, KD)

    mus = _make_gamma_final(B, KD)(bx, bu, ik, fk, d, c, a, mu, eps)
    return mus.reshape(B, K, D)
